# baseline (speedup 1.0000x reference)
import functools

import jax
import jax.numpy as jnp
from jax import lax
from jax.experimental import pallas as pl
from jax.experimental.pallas import tpu as pltpu
from jax.experimental.pallas import tpu_sc as plsc

N = 2048
E = 65536
D = 256
H = 8
HD = D // H
EB = 2048
NB = E // EB

_F32 = jnp.float32
_BF16 = jnp.bfloat16


def _bn(x, w, b, eps=1e-5):
    mean = jnp.mean(x, axis=0, keepdims=True)
    xc = x - mean
    var = jnp.mean(xc * xc, axis=0, keepdims=True)
    return xc / jnp.sqrt(var + eps) * w + b


def _node_mlp(x, n1w, n1b, n2w, n2b):
    xh = jax.nn.silu(jnp.dot(x, n1w, preferred_element_type=_F32) + n1b)
    return jnp.dot(xh, n2w, preferred_element_type=_F32) + n2b


def _prep_kernel(row_ref, col_ref, x_ref, pos_ref,
                 n1w_ref, n1b_ref, n2w_ref, n2b_ref,
                 e1w1_ref, e1b1_ref, e2w1_ref, e2b1_ref,
                 e1w2_ref, e1b2_ref, e2w2_ref, e2b2_ref,
                 e1_ref, e2_ref, xt1_ref, cnt_ref, cnt_s):
    i = pl.program_id(0)

    @pl.when(i == 0)
    def _init():
        xt1_ref[...] = _node_mlp(x_ref[...], n1w_ref[...], n1b_ref[...],
                                 n2w_ref[...], n2b_ref[...])
        cnt_s[...] = jnp.zeros_like(cnt_s)

    row = row_ref[...]
    col = col_ref[...]
    iota = jax.lax.broadcasted_iota(jnp.int32, (EB, N), 1)
    rowoh = (row == iota).astype(_BF16)
    coloh = (col == iota).astype(_BF16)
    cnt_s[0:1, :] += jnp.sum(coloh.astype(_F32), axis=0, keepdims=True)

    rel = jnp.dot(rowoh - coloh, pos_ref[...],
                  preferred_element_type=_F32)
    relT = jax.lax.transpose(rel[:, 0:8], (1, 0))
    rx, ry, rz = relT[0:1, :], relT[1:2, :], relT[2:3, :]
    s = rx * rx + ry * ry + rz * rz
    el = jnp.sqrt(s + 1e-12)
    scale = 1.0 / (jnp.sqrt(s) + 1e-10 * el)
    dx, dy, dz = rx * scale, ry * scale, rz * scale

    sphT = jnp.concatenate([
        jnp.full_like(dx, 0.28209479177387814),
        0.4886025119029199 * dx, 0.4886025119029199 * dy,
        0.4886025119029199 * dz,
        1.0925484305920792 * dx * dy,
        1.0925484305920792 * dy * dz,
        0.31539156525252005 * (3.0 * dz * dz - 1.0),
        1.0925484305920792 * dx * dz,
        0.5462742152960396 * (dx * dx - dy * dy),
        jnp.zeros((7, EB), _F32)], axis=0)

    for e1w, e1b, e2w, e2b, e_ref in (
            (e1w1_ref, e1b1_ref, e2w1_ref, e2b1_ref, e1_ref),
            (e1w2_ref, e1b2_ref, e2w2_ref, e2b2_ref, e2_ref)):
        h1 = jax.lax.dot_general(sphT, e1w[...], (((0,), (0,)), ((), ())),
                                 preferred_element_type=_F32) + e1b[...]
        h1 = jax.nn.silu(h1).astype(_BF16)
        e_ref[...] = (jnp.dot(h1, e2w[...], preferred_element_type=_F32)
                      + e2b[...])

    @pl.when(i == NB - 1)
    def _fin():
        cnt_ref[...] = cnt_s[0:1, :]


def _conv_kernel(with_xt2, row_ref, col_ref, e_ref, xin_ref, xt_ref,
                 cnt_ref,
                 oaw_ref, oxw_ref, o1b_ref, o2w_ref, o2b_ref,
                 bnw_ref, bnb_ref, nw_ref, nb_ref,
                 *rest):
    if with_xt2:
        (n1w_ref, n1b_ref, n2w_ref, n2b_ref,
         h_ref, xt2_ref, xtb_s, agg_s) = rest
    else:
        h_ref, xtb_s, agg_s = rest
    i = pl.program_id(0)

    @pl.when(i == 0)
    def _init():
        xtb_s[...] = xt_ref[...].astype(_BF16)
        agg_s[...] = jnp.zeros_like(agg_s)

    row = row_ref[...]
    col = col_ref[...]
    iota = jax.lax.broadcasted_iota(jnp.int32, (EB, N), 1)
    rowoh = (row == iota).astype(_BF16)
    coloh = (col == iota).astype(_BF16)

    xtg = jnp.dot(rowoh, xtb_s[...], preferred_element_type=_F32)
    msgs = (xtg * e_ref[...]).astype(_BF16)
    agg_s[...] += jax.lax.dot_general(
        coloh, msgs, (((0,), (0,)), ((), ())), preferred_element_type=_F32)

    @pl.when(i == NB - 1)
    def _finalize():
        cnt = jnp.maximum(cnt_ref[...], 1.0)
        agg = agg_s[...] / cnt
        g1 = jax.nn.silu(
            jnp.dot(agg, oaw_ref[...], preferred_element_type=_F32)
            + jnp.dot(xin_ref[...], oxw_ref[...], preferred_element_type=_F32)
            + o1b_ref[...])
        out = (jnp.dot(g1, o2w_ref[...], preferred_element_type=_F32)
               + o2b_ref[...])
        out = _bn(out, bnw_ref[...], bnb_ref[...])
        h = jax.nn.silu(_bn(out, nw_ref[...], nb_ref[...]))
        h_ref[...] = h
        if with_xt2:
            xt2_ref[...] = _node_mlp(h, n1w_ref[...], n1b_ref[...],
                                     n2w_ref[...], n2b_ref[...])


SC_NC = 2
SC_NS = 16
SC_L = 16
SC_C = 512


def _sc_conv_kernel(row_hbm, col_hbm, xt_hbm, e_hbm, out_hbm,
                    row_v, col_v, e_v, xts_v, agg_v):
    c = lax.axis_index("c")
    s = lax.axis_index("s")
    eph = E // SC_NC
    nch = eph // SC_C

    pltpu.sync_copy(xt_hbm.at[s], xts_v)
    zero = jnp.zeros((SC_L,), _F32)

    def zrow(r, carry):
        agg_v[r, :] = zero
        return carry
    lax.fori_loop(0, N, zrow, 0)

    def chunk(j, carry):
        ebase = c * eph + j * SC_C
        pltpu.sync_copy(row_hbm.at[pl.ds(ebase, SC_C)], row_v)
        pltpu.sync_copy(col_hbm.at[pl.ds(ebase, SC_C)], col_v)
        pltpu.sync_copy(e_hbm.at[s, pl.ds(ebase, SC_C)], e_v)

        def grp(g, carry2):
            row16 = row_v[pl.ds(g * 16, 16)]
            col16 = col_v[pl.ds(g * 16, 16)]
            for j16 in range(16):
                r = row16[j16]
                cc = col16[j16]
                msg = xts_v[r, :] * e_v[g * 16 + j16, :]
                agg_v[cc, :] = agg_v[cc, :] + msg
            return carry2
        lax.fori_loop(0, SC_C // 16, grp, 0)
        return carry
    lax.fori_loop(0, nch, chunk, 0)

    pltpu.sync_copy(agg_v, out_hbm.at[c, s])


def _sc_conv_call(row_f, col_f, xt16, e16):
    return pl.kernel(
        _sc_conv_kernel,
        mesh=plsc.VectorSubcoreMesh(core_axis_name="c", subcore_axis_name="s"),
        compiler_params=pltpu.CompilerParams(use_tc_tiling_on_sc=False),
        out_type=pltpu.HBM((SC_NC, SC_NS, N, SC_L), _F32),
        scratch_types=[
            pltpu.VMEM((SC_C,), jnp.int32),
            pltpu.VMEM((SC_C,), jnp.int32),
            pltpu.VMEM((SC_C, SC_L), _F32),
            pltpu.VMEM((N, SC_L), _F32),
            pltpu.VMEM((N, SC_L), _F32),
        ],
    )(row_f, col_f, xt16, e16)


def _fin_kernel(with_xt2, a0_ref, a1_ref, xin_ref, cnt_ref,
                oaw_ref, oxw_ref, o1b_ref, o2w_ref, o2b_ref,
                bnw_ref, bnb_ref, nw_ref, nb_ref, *rest):
    if with_xt2:
        (n1w_ref, n1b_ref, n2w_ref, n2b_ref, h_ref, xt2_ref) = rest
    else:
        (h_ref,) = rest
    cnt = jnp.maximum(cnt_ref[...], 1.0)
    agg = (a0_ref[...] + a1_ref[...]) / cnt
    g1 = jax.nn.silu(
        jnp.dot(agg, oaw_ref[...], preferred_element_type=_F32)
        + jnp.dot(xin_ref[...], oxw_ref[...], preferred_element_type=_F32)
        + o1b_ref[...])
    out = (jnp.dot(g1, o2w_ref[...], preferred_element_type=_F32)
           + o2b_ref[...])
    out = _bn(out, bnw_ref[...], bnb_ref[...])
    h = jax.nn.silu(_bn(out, nw_ref[...], nb_ref[...]))
    h_ref[...] = h
    if with_xt2:
        xt2_ref[...] = _node_mlp(h, n1w_ref[...], n1b_ref[...],
                                 n2w_ref[...], n2b_ref[...])


def _fin_call(aggp, xin, cnt_col, w, node_w):
    with_xt2 = node_w is not None
    allw = list(w) + (list(node_w) if with_xt2 else [])
    nout = 2 if with_xt2 else 1
    out_specs = [_full((N, D))] * nout
    out_shape = [jax.ShapeDtypeStruct((N, D), _F32)] * nout
    a0 = aggp[0].transpose(1, 0, 2).reshape(N, D)
    a1 = aggp[1].transpose(1, 0, 2).reshape(N, D)
    return pl.pallas_call(
        functools.partial(_fin_kernel, with_xt2),
        grid=(1,),
        in_specs=([_full((N, D)), _full((N, D)), _full((N, D)),
                   _full((N, 1))] + [_full(a.shape) for a in allw]),
        out_specs=out_specs if with_xt2 else out_specs[0],
        out_shape=out_shape if with_xt2 else out_shape[0],
    )(a0, a1, xin, cnt_col, *allw)


def _attn_kernel(h_ref, wq_ref, bq_ref, wk_ref, bk_ref, wv_ref, bv_ref,
                 wo_ref, bo_ref, out_ref):
    i = pl.program_id(0)
    h = h_ref[...].astype(_BF16)
    q = jnp.dot(h, wq_ref[0], preferred_element_type=_F32) + bq_ref[0]
    k = jnp.dot(h, wk_ref[0], preferred_element_type=_F32) + bk_ref[0]
    v = jnp.dot(h, wv_ref[0], preferred_element_type=_F32) + bv_ref[0]
    s = jax.lax.dot_general(q.astype(_BF16), k.astype(_BF16),
                            (((1,), (1,)), ((), ())),
                            preferred_element_type=_F32) * (HD ** -0.5)
    p = jax.nn.softmax(s, axis=-1)
    o = jnp.dot(p.astype(_BF16), v.astype(_BF16),
                preferred_element_type=_F32)
    contrib = jnp.dot(o.astype(_BF16), wo_ref[0], preferred_element_type=_F32)

    @pl.when(i == 0)
    def _first():
        out_ref[...] = contrib + bo_ref[...]

    @pl.when(i > 0)
    def _rest():
        out_ref[...] += contrib


def _full(shape):
    return pl.BlockSpec(shape, lambda i: (0,) * len(shape))


def _eblk():
    return pl.BlockSpec((EB, 1), lambda i: (i, 0))


def _erow():
    return pl.BlockSpec((EB, D), lambda i: (i, 0))


def _prep_call(row, col, x, pos_p, w):
    in_specs = [_eblk(), _eblk(), _full((N, D)), _full((N, 128))]
    in_specs += [_full(a.shape) for a in w]
    return pl.pallas_call(
        _prep_kernel,
        grid=(NB,),
        in_specs=in_specs,
        out_specs=[_erow(), _erow(), _full((N, D)), _full((1, N))],
        out_shape=[jax.ShapeDtypeStruct((E, D), _F32),
                   jax.ShapeDtypeStruct((E, D), _F32),
                   jax.ShapeDtypeStruct((N, D), _F32),
                   jax.ShapeDtypeStruct((1, N), _F32)],
        scratch_shapes=[pltpu.VMEM((8, N), _F32)],
    )(row, col, x, pos_p, *w)


def _conv_call(row, col, e_all, xin, xt, cnt_col, w, node_w):
    with_xt2 = node_w is not None
    allw = list(w) + (list(node_w) if with_xt2 else [])
    in_specs = ([_eblk(), _eblk(), _erow(), _full((N, D)), _full((N, D)),
                 _full((N, 1))] + [_full(a.shape) for a in allw])
    out_specs = [_full((N, D))] * (2 if with_xt2 else 1)
    out_shape = [jax.ShapeDtypeStruct((N, D), _F32)] * (2 if with_xt2 else 1)
    res = pl.pallas_call(
        functools.partial(_conv_kernel, with_xt2),
        grid=(NB,),
        in_specs=in_specs,
        out_specs=out_specs if with_xt2 else out_specs[0],
        out_shape=out_shape if with_xt2 else out_shape[0],
        scratch_shapes=[pltpu.VMEM((N, D), _BF16),
                        pltpu.VMEM((N, D), _F32)],
    )(row, col, e_all, xin, xt, cnt_col, *allw)
    return res


def _attn_call(h, p):
    wq = p["attn_q_w"].reshape(D, H, HD).transpose(1, 0, 2).astype(_BF16)
    wk = p["attn_k_w"].reshape(D, H, HD).transpose(1, 0, 2).astype(_BF16)
    wv = p["attn_v_w"].reshape(D, H, HD).transpose(1, 0, 2).astype(_BF16)
    bq = p["attn_q_b"].reshape(H, 1, HD)
    bk = p["attn_k_b"].reshape(H, 1, HD)
    bv = p["attn_v_b"].reshape(H, 1, HD)
    wo = p["attn_o_w"].reshape(H, HD, D).astype(_BF16)
    bo = p["attn_o_b"].reshape(1, D)
    hw = pl.BlockSpec((1, D, HD), lambda i: (i, 0, 0))
    hb = pl.BlockSpec((1, 1, HD), lambda i: (i, 0, 0))
    ho = pl.BlockSpec((1, HD, D), lambda i: (i, 0, 0))
    return pl.pallas_call(
        _attn_kernel,
        grid=(H,),
        in_specs=[_full((N, D)), hw, hb, hw, hb, hw, hb, ho, _full((1, D))],
        out_specs=_full((N, D)),
        out_shape=jax.ShapeDtypeStruct((N, D), _F32),
    )(h, wq, bq, wk, bk, wv, bv, wo, bo)


def _pad16(w):
    return jnp.pad(w, ((0, 16 - w.shape[0]), (0, 0)))


def kernel(x, edge_index, edge_attr, pos, params):
    p = params
    b = lambda name: p[name + "_b"].reshape(1, D)
    row = edge_index[0].reshape(E, 1)
    col = edge_index[1].reshape(E, 1)
    pos_p = jnp.pad(pos, ((0, 0), (0, 128 - pos.shape[1]))).astype(_BF16)

    prep_w = (p["conv1_node1_w"], b("conv1_node1"),
              p["conv1_node2_w"], b("conv1_node2"),
              _pad16(p["conv1_edge1_w"]), b("conv1_edge1"),
              p["conv1_edge2_w"].astype(_BF16), b("conv1_edge2"),
              _pad16(p["conv2_edge1_w"]), b("conv2_edge1"),
              p["conv2_edge2_w"].astype(_BF16), b("conv2_edge2"))
    e1_all, e2_all, xt1, cnt = _prep_call(row, col, x, pos_p, prep_w)
    cnt_col = cnt.reshape(N, 1)

    def conv_w(prefix):
        o1w = p[prefix + "_out1_w"]
        return (o1w[:D], o1w[D:], b(prefix + "_out1"),
                p[prefix + "_out2_w"], b(prefix + "_out2"),
                p[prefix + "_bn_w"].reshape(1, D), p[prefix + "_bn_b"].reshape(1, D))

    bn1 = (p["norm1_w"].reshape(1, D), p["norm1_b"].reshape(1, D))
    bn2 = (p["norm2_w"].reshape(1, D), p["norm2_b"].reshape(1, D))
    node2_w = (p["conv2_node1_w"], b("conv2_node1"),
               p["conv2_node2_w"], b("conv2_node2"))

    row_f = edge_index[0].reshape(E)
    col_f = edge_index[1].reshape(E)

    def col_major(a, n):
        return a.reshape(n, SC_NS, SC_L).transpose(1, 0, 2)

    aggp1 = _sc_conv_call(row_f, col_f, col_major(xt1, N),
                          col_major(e1_all, E))
    h1, xt2 = _fin_call(aggp1, x, cnt_col, conv_w("conv1") + bn1, node2_w)
    aggp2 = _sc_conv_call(row_f, col_f, col_major(xt2, N),
                          col_major(e2_all, E))
    h2 = _fin_call(aggp2, h1, cnt_col, conv_w("conv2") + bn2, None)
    return _attn_call(h2, p)


# speedup vs baseline: 2.0034x; 2.0034x over previous
import functools

import jax
import jax.numpy as jnp
from jax.experimental import pallas as pl
from jax.experimental.pallas import tpu as pltpu

N = 2048
E = 65536
D = 256
H = 8
HD = D // H
EB = 2048
NB = E // EB

_F32 = jnp.float32
_BF16 = jnp.bfloat16


def _bn(x, w, b, eps=1e-5):
    mean = jnp.mean(x, axis=0, keepdims=True)
    xc = x - mean
    var = jnp.mean(xc * xc, axis=0, keepdims=True)
    return xc / jnp.sqrt(var + eps) * w + b


def _node_mlp(x, n1w, n1b, n2w, n2b):
    xh = jax.nn.silu(jnp.dot(x, n1w, preferred_element_type=_F32) + n1b)
    return jnp.dot(xh, n2w, preferred_element_type=_F32) + n2b


def _prep_kernel(row_ref, col_ref, x_ref, pos_ref,
                 n1w_ref, n1b_ref, n2w_ref, n2b_ref,
                 e1w1_ref, e1b1_ref, e2w1_ref, e2b1_ref,
                 e1w2_ref, e1b2_ref, e2w2_ref, e2b2_ref,
                 e1_ref, e2_ref, xt1_ref, cnt_ref, cnt_s):
    i = pl.program_id(0)

    @pl.when(i == 0)
    def _init():
        xt1_ref[...] = _node_mlp(x_ref[...], n1w_ref[...], n1b_ref[...],
                                 n2w_ref[...], n2b_ref[...])
        cnt_s[...] = jnp.zeros_like(cnt_s)

    row = row_ref[...]
    col = col_ref[...]
    iota = jax.lax.broadcasted_iota(jnp.int32, (EB, N), 1)
    rowoh = (row == iota).astype(_BF16)
    coloh = (col == iota).astype(_BF16)
    cnt_s[0:1, :] += jnp.sum(coloh.astype(_F32), axis=0, keepdims=True)

    rel = jnp.dot(rowoh - coloh, pos_ref[...],
                  preferred_element_type=_F32)
    relT = jax.lax.transpose(rel[:, 0:8], (1, 0))
    rx, ry, rz = relT[0:1, :], relT[1:2, :], relT[2:3, :]
    s = rx * rx + ry * ry + rz * rz
    el = jnp.sqrt(s + 1e-12)
    scale = 1.0 / (jnp.sqrt(s) + 1e-10 * el)
    dx, dy, dz = rx * scale, ry * scale, rz * scale

    sphT = jnp.concatenate([
        jnp.full_like(dx, 0.28209479177387814),
        0.4886025119029199 * dx, 0.4886025119029199 * dy,
        0.4886025119029199 * dz,
        1.0925484305920792 * dx * dy,
        1.0925484305920792 * dy * dz,
        0.31539156525252005 * (3.0 * dz * dz - 1.0),
        1.0925484305920792 * dx * dz,
        0.5462742152960396 * (dx * dx - dy * dy),
        jnp.zeros((7, EB), _F32)], axis=0)

    for e1w, e1b, e2w, e2b, e_ref in (
            (e1w1_ref, e1b1_ref, e2w1_ref, e2b1_ref, e1_ref),
            (e1w2_ref, e1b2_ref, e2w2_ref, e2b2_ref, e2_ref)):
        h1 = jax.lax.dot_general(sphT, e1w[...], (((0,), (0,)), ((), ())),
                                 preferred_element_type=_F32) + e1b[...]
        h1 = jax.nn.silu(h1).astype(_BF16)
        e_ref[...] = (jnp.dot(h1, e2w[...], preferred_element_type=_F32)
                      + e2b[...])

    @pl.when(i == NB - 1)
    def _fin():
        cnt_ref[...] = cnt_s[0:1, :]


def _conv_kernel(with_xt2, row_ref, colt_ref, e_ref, xin_ref, xt_ref,
                 cnt_ref,
                 oaw_ref, oxw_ref, o1b_ref, o2w_ref, o2b_ref,
                 bnw_ref, bnb_ref, nw_ref, nb_ref,
                 *rest):
    if with_xt2:
        (n1w_ref, n1b_ref, n2w_ref, n2b_ref,
         h_ref, xt2_ref, xtb_s, agg_s) = rest
    else:
        h_ref, xtb_s, agg_s = rest
    i = pl.program_id(0)

    @pl.when(i == 0)
    def _init():
        xtb_s[...] = xt_ref[...].astype(_BF16)
        agg_s[...] = jnp.zeros_like(agg_s)

    row = row_ref[...]
    colt = colt_ref[0]
    iota = jax.lax.broadcasted_iota(jnp.int32, (EB, N), 1)
    iota_t = jax.lax.broadcasted_iota(jnp.int32, (N, EB), 0)
    rowoh = (row == iota).astype(_BF16)
    colohT = (colt == iota_t).astype(_BF16)

    xtg = jnp.dot(rowoh, xtb_s[...], preferred_element_type=_F32)
    msgs = (xtg * e_ref[...]).astype(_BF16)
    agg_s[...] += jnp.dot(colohT, msgs, preferred_element_type=_F32)

    @pl.when(i == NB - 1)
    def _finalize():
        cnt = jnp.maximum(cnt_ref[...], 1.0)
        agg = agg_s[...] / cnt
        g1 = jax.nn.silu(
            jnp.dot(agg, oaw_ref[...], preferred_element_type=_F32)
            + jnp.dot(xin_ref[...], oxw_ref[...], preferred_element_type=_F32)
            + o1b_ref[...])
        out = (jnp.dot(g1, o2w_ref[...], preferred_element_type=_F32)
               + o2b_ref[...])
        out = _bn(out, bnw_ref[...], bnb_ref[...])
        h = jax.nn.silu(_bn(out, nw_ref[...], nb_ref[...]))
        h_ref[...] = h
        if with_xt2:
            xt2_ref[...] = _node_mlp(h, n1w_ref[...], n1b_ref[...],
                                     n2w_ref[...], n2b_ref[...])


def _attn_kernel(h_ref, wq_ref, bq_ref, wk_ref, bk_ref, wv_ref, bv_ref,
                 wo_ref, bo_ref, out_ref):
    i = pl.program_id(0)
    h = h_ref[...].astype(_BF16)
    q = jnp.dot(h, wq_ref[0], preferred_element_type=_F32) + bq_ref[0]
    k = jnp.dot(h, wk_ref[0], preferred_element_type=_F32) + bk_ref[0]
    v = jnp.dot(h, wv_ref[0], preferred_element_type=_F32) + bv_ref[0]
    s = jax.lax.dot_general(q.astype(_BF16), k.astype(_BF16),
                            (((1,), (1,)), ((), ())),
                            preferred_element_type=_F32) * (HD ** -0.5)
    p = jax.nn.softmax(s, axis=-1)
    o = jnp.dot(p.astype(_BF16), v.astype(_BF16),
                preferred_element_type=_F32)
    contrib = jnp.dot(o.astype(_BF16), wo_ref[0], preferred_element_type=_F32)

    @pl.when(i == 0)
    def _first():
        out_ref[...] = contrib + bo_ref[...]

    @pl.when(i > 0)
    def _rest():
        out_ref[...] += contrib


def _full(shape):
    return pl.BlockSpec(shape, lambda i: (0,) * len(shape))


def _eblk():
    return pl.BlockSpec((EB, 1), lambda i: (i, 0))


def _erow():
    return pl.BlockSpec((EB, D), lambda i: (i, 0))


def _prep_call(row, col, x, pos_p, w):
    in_specs = [_eblk(), _eblk(), _full((N, D)), _full((N, 128))]
    in_specs += [_full(a.shape) for a in w]
    return pl.pallas_call(
        _prep_kernel,
        grid=(NB,),
        in_specs=in_specs,
        out_specs=[_erow(), _erow(), _full((N, D)), _full((1, N))],
        out_shape=[jax.ShapeDtypeStruct((E, D), _F32),
                   jax.ShapeDtypeStruct((E, D), _F32),
                   jax.ShapeDtypeStruct((N, D), _F32),
                   jax.ShapeDtypeStruct((1, N), _F32)],
        scratch_shapes=[pltpu.VMEM((8, N), _F32)],
    )(row, col, x, pos_p, *w)


def _conv_call(row, colt, e_all, xin, xt, cnt_col, w, node_w):
    with_xt2 = node_w is not None
    allw = list(w) + (list(node_w) if with_xt2 else [])
    in_specs = ([_eblk(), pl.BlockSpec((1, 1, EB), lambda i: (i, 0, 0)),
                 _erow(), _full((N, D)), _full((N, D)),
                 _full((N, 1))] + [_full(a.shape) for a in allw])
    out_specs = [_full((N, D))] * (2 if with_xt2 else 1)
    out_shape = [jax.ShapeDtypeStruct((N, D), _F32)] * (2 if with_xt2 else 1)
    res = pl.pallas_call(
        functools.partial(_conv_kernel, with_xt2),
        grid=(NB,),
        in_specs=in_specs,
        out_specs=out_specs if with_xt2 else out_specs[0],
        out_shape=out_shape if with_xt2 else out_shape[0],
        scratch_shapes=[pltpu.VMEM((N, D), _BF16),
                        pltpu.VMEM((N, D), _F32)],
    )(row, colt, e_all, xin, xt, cnt_col, *allw)
    return res


def _attn_call(h, p):
    wq = p["attn_q_w"].reshape(D, H, HD).transpose(1, 0, 2).astype(_BF16)
    wk = p["attn_k_w"].reshape(D, H, HD).transpose(1, 0, 2).astype(_BF16)
    wv = p["attn_v_w"].reshape(D, H, HD).transpose(1, 0, 2).astype(_BF16)
    bq = p["attn_q_b"].reshape(H, 1, HD)
    bk = p["attn_k_b"].reshape(H, 1, HD)
    bv = p["attn_v_b"].reshape(H, 1, HD)
    wo = p["attn_o_w"].reshape(H, HD, D).astype(_BF16)
    bo = p["attn_o_b"].reshape(1, D)
    hw = pl.BlockSpec((1, D, HD), lambda i: (i, 0, 0))
    hb = pl.BlockSpec((1, 1, HD), lambda i: (i, 0, 0))
    ho = pl.BlockSpec((1, HD, D), lambda i: (i, 0, 0))
    return pl.pallas_call(
        _attn_kernel,
        grid=(H,),
        in_specs=[_full((N, D)), hw, hb, hw, hb, hw, hb, ho, _full((1, D))],
        out_specs=_full((N, D)),
        out_shape=jax.ShapeDtypeStruct((N, D), _F32),
    )(h, wq, bq, wk, bk, wv, bv, wo, bo)


def _pad16(w):
    return jnp.pad(w, ((0, 16 - w.shape[0]), (0, 0)))


def kernel(x, edge_index, edge_attr, pos, params):
    p = params
    b = lambda name: p[name + "_b"].reshape(1, D)
    row = edge_index[0].reshape(E, 1)
    col = edge_index[1].reshape(E, 1)
    pos_p = jnp.pad(pos, ((0, 0), (0, 128 - pos.shape[1]))).astype(_BF16)

    prep_w = (p["conv1_node1_w"], b("conv1_node1"),
              p["conv1_node2_w"], b("conv1_node2"),
              _pad16(p["conv1_edge1_w"]), b("conv1_edge1"),
              p["conv1_edge2_w"].astype(_BF16), b("conv1_edge2"),
              _pad16(p["conv2_edge1_w"]), b("conv2_edge1"),
              p["conv2_edge2_w"].astype(_BF16), b("conv2_edge2"))
    e1_all, e2_all, xt1, cnt = _prep_call(row, col, x, pos_p, prep_w)
    cnt_col = cnt.reshape(N, 1)

    def conv_w(prefix):
        o1w = p[prefix + "_out1_w"]
        return (o1w[:D], o1w[D:], b(prefix + "_out1"),
                p[prefix + "_out2_w"], b(prefix + "_out2"),
                p[prefix + "_bn_w"].reshape(1, D), p[prefix + "_bn_b"].reshape(1, D))

    bn1 = (p["norm1_w"].reshape(1, D), p["norm1_b"].reshape(1, D))
    bn2 = (p["norm2_w"].reshape(1, D), p["norm2_b"].reshape(1, D))
    node2_w = (p["conv2_node1_w"], b("conv2_node1"),
               p["conv2_node2_w"], b("conv2_node2"))

    colt = edge_index[1].reshape(NB, 1, EB)
    h1, xt2 = _conv_call(row, colt, e1_all, x, xt1, cnt_col,
                         conv_w("conv1") + bn1, node2_w)
    h2 = _conv_call(row, colt, e2_all, h1, xt2, cnt_col,
                    conv_w("conv2") + bn2, None)
    return _attn_call(h2, p)
